# baseline (speedup 1.0000x reference)
import jax
import jax.numpy as jnp
from jax import lax
from jax.experimental import pallas as pl
from jax.experimental.pallas import tpu as pltpu
from jax.experimental.pallas import tpu_sc as plsc

_B, _D, _H, _W = 8, 48, 160, 320
_SC_H = 16
_TC_H = _H - _SC_H
_ROWS = 8
_CPX = _ROWS * _W
_DQ = 12
_DHALF = _D // 2
_NQ = _DHALF // _DQ
_LANES = 16
_NPAIR = 8
_TC_BH = 48


def _top2_update(v, df, max1, max2, i1, i2):
    gt1 = v > max1
    gt2 = v > max2
    i2 = jnp.where(gt1, i1, jnp.where(gt2, df, i2))
    max2 = jnp.where(gt1, max1, jnp.where(gt2, v, max2))
    i1 = jnp.where(gt1, df, i1)
    max1 = jnp.where(gt1, v, max1)
    return max1, max2, i1, i2


def _sc_body(cost_hbm, out_hbm, buf, st, st2, obuf, shared, isem0, isem1, osem):
    c = lax.axis_index("c")
    s = lax.axis_index("s")
    m = s // 2
    half = s % 2
    k = c * _NPAIR + m
    b = k // 2
    orow0 = (k % 2) * _ROWS
    row0 = _TC_H + orow0
    d0 = half * _DHALF
    isems = (isem0, isem1)

    def in_copy(q, slot):
        src = cost_hbm.at[
            b, pl.ds(d0 + q * _DQ, _DQ), pl.ds(row0, _ROWS), :
        ]
        return pltpu.make_async_copy(src, buf.at[slot], isems[slot])

    def compute_tile(q, slot):
        def body(ww, _):
            w16 = ww * _LANES
            for hh in range(_ROWS):
                s16 = hh * _W + w16
                if q == 0:
                    v = buf[slot, 0, hh, pl.ds(w16, _LANES)]
                    max1 = v
                    i1 = jnp.full((_LANES,), jnp.float32(d0))
                    max2 = jnp.full((_LANES,), -jnp.inf, jnp.float32)
                    i2 = jnp.full((_LANES,), jnp.float32(d0))
                    dds = range(1, _DQ)
                else:
                    max1 = st[0, pl.ds(s16, _LANES)]
                    max2 = st[1, pl.ds(s16, _LANES)]
                    i1 = st[2, pl.ds(s16, _LANES)]
                    i2 = st[3, pl.ds(s16, _LANES)]
                    dds = range(_DQ)
                for dd in dds:
                    v = buf[slot, dd, hh, pl.ds(w16, _LANES)]
                    df = jnp.float32(d0 + q * _DQ + dd)
                    max1, max2, i1, i2 = _top2_update(v, df, max1, max2, i1, i2)
                st[0, pl.ds(s16, _LANES)] = max1
                st[1, pl.ds(s16, _LANES)] = max2
                st[2, pl.ds(s16, _LANES)] = i1
                st[3, pl.ds(s16, _LANES)] = i2
            return 0

        lax.fori_loop(0, _W // _LANES, body, 0)

    in_copy(0, 0).start()
    for q in range(_NQ):
        slot = q % 2
        in_copy(q, slot).wait()
        if q + 1 < _NQ:
            in_copy(q + 1, 1 - slot).start()
        compute_tile(q, slot)

    @pl.when(half == 1)
    def _():
        pltpu.sync_copy(st, shared.at[m])

    plsc.subcore_barrier()

    @pl.when(half == 0)
    def _():
        pltpu.sync_copy(shared.at[m], st2)

        def merge(j, _):
            j16 = j * _LANES
            a1 = st[0, pl.ds(j16, _LANES)]
            a2 = st[1, pl.ds(j16, _LANES)]
            ai1 = st[2, pl.ds(j16, _LANES)]
            ai2 = st[3, pl.ds(j16, _LANES)]
            b1 = st2[0, pl.ds(j16, _LANES)]
            b2 = st2[1, pl.ds(j16, _LANES)]
            bi1 = st2[2, pl.ds(j16, _LANES)]
            bi2 = st2[3, pl.ds(j16, _LANES)]
            gt = b1 > a1
            m1 = jnp.where(gt, b1, a1)
            mi1 = jnp.where(gt, bi1, ai1)
            lv = jnp.where(gt, a1, b1)
            li = jnp.where(gt, ai1, bi1)
            ov = jnp.where(gt, b2, a2)
            oi = jnp.where(gt, bi2, ai2)
            take_o = (ov > lv) | ((ov == lv) & (oi < li))
            m2 = jnp.where(take_o, ov, lv)
            mi2 = jnp.where(take_o, oi, li)
            e = jnp.exp(m2 - m1)
            hh = j // (_W // _LANES)
            w16 = (j % (_W // _LANES)) * _LANES
            obuf[hh, pl.ds(w16, _LANES)] = (mi1 + mi2 * e) / (1.0 + e)
            return 0

        lax.fori_loop(0, _CPX // _LANES, merge, 0)
        dst = out_hbm.at[b, 0, pl.ds(orow0, _ROWS), :]
        pltpu.make_async_copy(obuf, dst, osem).start()
        pltpu.make_async_copy(obuf, dst, osem).wait()


def _tc_body(x_ref, o_ref):
    x = x_ref[0]
    max1 = x[0]
    shape = max1.shape
    i1 = jnp.zeros(shape, jnp.float32)
    max2 = jnp.full(shape, -jnp.inf, jnp.float32)
    i2 = jnp.zeros(shape, jnp.float32)
    for d in range(1, _D):
        max1, max2, i1, i2 = _top2_update(
            x[d], jnp.float32(d), max1, max2, i1, i2
        )
    e = jnp.exp(max2 - max1)
    o_ref[0, 0] = (i1 + i2 * e) / (1.0 + e)


@jax.jit
def kernel(cost):
    mesh = plsc.VectorSubcoreMesh(
        core_axis_name="c", subcore_axis_name="s", num_cores=2, num_subcores=16
    )
    disp_sc = pl.kernel(
        _sc_body,
        out_type=jax.ShapeDtypeStruct((_B, 1, _SC_H, _W), jnp.float32),
        mesh=mesh,
        scratch_types=[
            pltpu.VMEM((2, _DQ, _ROWS, _W), jnp.float32),
            pltpu.VMEM((4, _CPX), jnp.float32),
            pltpu.VMEM((4, _CPX), jnp.float32),
            pltpu.VMEM((_ROWS, _W), jnp.float32),
            pltpu.VMEM_SHARED((_NPAIR, 4, _CPX), jnp.float32),
            pltpu.SemaphoreType.DMA,
            pltpu.SemaphoreType.DMA,
            pltpu.SemaphoreType.DMA,
        ],
    )(cost)

    disp_tc = pl.pallas_call(
        _tc_body,
        grid=(_B, _TC_H // _TC_BH),
        in_specs=[
            pl.BlockSpec((1, _D, _TC_BH, _W), lambda i, j: (i, 0, j, 0))
        ],
        out_specs=pl.BlockSpec((1, 1, _TC_BH, _W), lambda i, j: (i, 0, j, 0)),
        out_shape=jax.ShapeDtypeStruct((_B, 1, _H, _W), jnp.float32),
    )(cost)

    return lax.dynamic_update_slice(disp_tc, disp_sc, (0, 0, _TC_H, 0))


# speedup vs baseline: 1.1270x; 1.1270x over previous
import jax
import jax.numpy as jnp
from jax import lax
from jax.experimental import pallas as pl
from jax.experimental.pallas import tpu as pltpu
from jax.experimental.pallas import tpu_sc as plsc

_B, _D, _H, _W = 8, 48, 160, 320
_SC_H = 32
_TC_H = _H - _SC_H
_ROWS = 8
_DQ = 12
_NQ = _D // _DQ
_LANES = 16
_WPB = _SC_H // _ROWS
_TC_BH = 128


def _top2_update(v, df, max1, max2, i1, i2):
    gt1 = v > max1
    gt2 = v > max2
    i2 = jnp.where(gt1, i1, jnp.where(gt2, df, i2))
    max2 = jnp.where(gt1, max1, jnp.where(gt2, v, max2))
    i1 = jnp.where(gt1, df, i1)
    max1 = jnp.where(gt1, v, max1)
    return max1, max2, i1, i2


def _sc_body(cost_hbm, out_hbm, buf, st, obuf, isem0, isem1, osem):
    nc = 2
    wid = lax.axis_index("s") * nc + lax.axis_index("c")
    b = wid // _WPB
    orow0 = (wid % _WPB) * _ROWS
    row0 = _TC_H + orow0
    isems = (isem0, isem1)

    def in_copy(q, slot):
        src = cost_hbm.at[b, pl.ds(q * _DQ, _DQ), pl.ds(row0, _ROWS), :]
        return pltpu.make_async_copy(src, buf.at[slot], isems[slot])

    def compute_tile(q, slot):
        def body(ww, _):
            w16 = ww * _LANES
            for hh in range(_ROWS):
                s16 = hh * _W + w16
                if q == 0:
                    v = buf[slot, 0, hh, pl.ds(w16, _LANES)]
                    max1 = v
                    i1 = jnp.zeros((_LANES,), jnp.float32)
                    max2 = jnp.full((_LANES,), -jnp.inf, jnp.float32)
                    i2 = jnp.zeros((_LANES,), jnp.float32)
                    dds = range(1, _DQ)
                else:
                    max1 = st[0, pl.ds(s16, _LANES)]
                    max2 = st[1, pl.ds(s16, _LANES)]
                    i1 = st[2, pl.ds(s16, _LANES)]
                    i2 = st[3, pl.ds(s16, _LANES)]
                    dds = range(_DQ)
                for dd in dds:
                    v = buf[slot, dd, hh, pl.ds(w16, _LANES)]
                    df = jnp.float32(q * _DQ + dd)
                    max1, max2, i1, i2 = _top2_update(v, df, max1, max2, i1, i2)
                if q == _NQ - 1:
                    e = jnp.exp(max2 - max1)
                    obuf[hh, pl.ds(w16, _LANES)] = (i1 + i2 * e) / (1.0 + e)
                else:
                    st[0, pl.ds(s16, _LANES)] = max1
                    st[1, pl.ds(s16, _LANES)] = max2
                    st[2, pl.ds(s16, _LANES)] = i1
                    st[3, pl.ds(s16, _LANES)] = i2
            return 0

        lax.fori_loop(0, _W // _LANES, body, 0)

    in_copy(0, 0).start()
    for q in range(_NQ):
        slot = q % 2
        in_copy(q, slot).wait()
        if q + 1 < _NQ:
            in_copy(q + 1, 1 - slot).start()
        compute_tile(q, slot)
    dst = out_hbm.at[b, 0, pl.ds(orow0, _ROWS), :]
    pltpu.make_async_copy(obuf, dst, osem).start()
    pltpu.make_async_copy(obuf, dst, osem).wait()


def _tc_body(x_ref, o_ref):
    x = x_ref[0]
    max1 = x[0]
    shape = max1.shape
    i1 = jnp.zeros(shape, jnp.float32)
    max2 = jnp.full(shape, -jnp.inf, jnp.float32)
    i2 = jnp.zeros(shape, jnp.float32)
    for d in range(1, _D):
        max1, max2, i1, i2 = _top2_update(
            x[d], jnp.float32(d), max1, max2, i1, i2
        )
    e = jnp.exp(max2 - max1)
    o_ref[0, 0] = (i1 + i2 * e) / (1.0 + e)


@jax.jit
def kernel(cost):
    mesh = plsc.VectorSubcoreMesh(
        core_axis_name="c", subcore_axis_name="s", num_cores=2, num_subcores=16
    )
    disp_sc = pl.kernel(
        _sc_body,
        out_type=jax.ShapeDtypeStruct((_B, 1, _SC_H, _W), jnp.float32),
        mesh=mesh,
        scratch_types=[
            pltpu.VMEM((2, _DQ, _ROWS, _W), jnp.float32),
            pltpu.VMEM((4, _ROWS * _W), jnp.float32),
            pltpu.VMEM((_ROWS, _W), jnp.float32),
            pltpu.SemaphoreType.DMA,
            pltpu.SemaphoreType.DMA,
            pltpu.SemaphoreType.DMA,
        ],
    )(cost)

    disp_tc = pl.pallas_call(
        _tc_body,
        grid=(_B, _TC_H // _TC_BH),
        in_specs=[
            pl.BlockSpec((1, _D, _TC_BH, _W), lambda i, j: (i, 0, j, 0))
        ],
        out_specs=pl.BlockSpec((1, 1, _TC_BH, _W), lambda i, j: (i, 0, j, 0)),
        out_shape=jax.ShapeDtypeStruct((_B, 1, _H, _W), jnp.float32),
    )(cost)

    return lax.dynamic_update_slice(disp_tc, disp_sc, (0, 0, _TC_H, 0))
